# Initial kernel scaffold; baseline (speedup 1.0000x reference)
import jax
import jax.numpy as jnp
from jax.experimental import pallas as pl


def kernel(x, pos_table):
    raise NotImplementedError("write your pallas kernel here")


# speedup vs baseline: 1.5449x; 1.5449x over previous
import jax
import jax.numpy as jnp
from jax.experimental import pallas as pl

B, S, D = 4, 4096, 1024
TS = 512


def _add_kernel(x_ref, emb_ref, out_ref):
    out_ref[...] = x_ref[...] + emb_ref[...][None, :, :]


def kernel(x, pos_table):
    emb = pos_table[:S]
    grid = (S // TS,)
    return pl.pallas_call(
        _add_kernel,
        grid=grid,
        in_specs=[
            pl.BlockSpec((B, TS, D), lambda i: (0, i, 0)),
            pl.BlockSpec((TS, D), lambda i: (i, 0)),
        ],
        out_specs=pl.BlockSpec((B, TS, D), lambda i: (0, i, 0)),
        out_shape=jax.ShapeDtypeStruct((B, S, D), x.dtype),
    )(x, emb)


# speedup vs baseline: 1.9732x; 1.2773x over previous
import jax
import jax.numpy as jnp
from jax.experimental import pallas as pl

B, S, D = 4, 4096, 1024
TS = 512


def _add_kernel(x_ref, emb_ref, out_ref):
    out_ref[...] = x_ref[...] + emb_ref[...][None, :, :]


def kernel(x, pos_table):
    grid = (S // TS,)
    return pl.pallas_call(
        _add_kernel,
        grid=grid,
        in_specs=[
            pl.BlockSpec((B, TS, D), lambda i: (0, i, 0)),
            pl.BlockSpec((TS, D), lambda i: (i, 0)),
        ],
        out_specs=pl.BlockSpec((B, TS, D), lambda i: (0, i, 0)),
        out_shape=jax.ShapeDtypeStruct((B, S, D), x.dtype),
    )(x, pos_table)
